# baseline (speedup 1.0000x reference)
import functools

import jax
import jax.numpy as jnp
from jax import lax
from jax.experimental import pallas as pl
from jax.experimental.pallas import tpu as pltpu
from jax.experimental.pallas import tpu_sc as plsc

N = 10000
E = 160000
X = 128
H = 128
M = 7
T = 16

NCHUNK = 20
NPAD = 10240
CHUNK = NPAD // NCHUNK
CH_PER_SC = NCHUNK // 2
ACC_ROWS = (M + 1) * CHUNK
ACC_ALLOC = ACC_ROWS + 512
EPT = E // 16
EB = 2000
NG16 = EB // 16
G = 64
ZR = 16
TPKN = N // 4 + 16


def _tc_pre_body(x_ref, wiou_ref, biou_ref, wf_ref, bf_ref, uf_ref,
                 xiou_ref, xf_ref, h_ref, c_ref, ufn_ref):
    xb = x_ref[...]
    xiou = jnp.dot(xb, wiou_ref[...], preferred_element_type=jnp.float32) + biou_ref[...]
    xf = jnp.dot(xb, wf_ref[...], preferred_element_type=jnp.float32) + bf_ref[...]
    i = jax.nn.sigmoid(xiou[:, :H])
    o = jax.nn.sigmoid(xiou[:, H:2 * H])
    u = jnp.tanh(xiou[:, 2 * H:])
    c1 = i * u
    h1 = o * jnp.tanh(c1)
    xiou_ref[...] = xiou
    xf_ref[...] = xf
    h_ref[...] = h1
    c_ref[...] = c1
    for m in range(M):
        ufn_ref[m] = jnp.dot(h1, uf_ref[m], preferred_element_type=jnp.float32)


def _tc_post_body(sums_ref, uiou_ref, xiou_ref, h_ref, c_ref):
    acc = xiou_ref[...]
    for m in range(M):
        acc = acc + jnp.dot(sums_ref[m + 1], uiou_ref[m], preferred_element_type=jnp.float32)
    i = jax.nn.sigmoid(acc[:, :H])
    o = jax.nn.sigmoid(acc[:, H:2 * H])
    u = jnp.tanh(acc[:, 2 * H:])
    c2 = i * u + sums_ref[0]
    h2 = o * jnp.tanh(c2)
    h_ref[...] = h2
    c_ref[...] = c2


def _sc_edge_body(esrc, edst, tpk, rel, htab, ctab, uftab, xftab,
                  out,
                  tpk_v, rel_v, epk, cpk, zbuf,
                  h_b0, c_b0, z_b0, h_b1, c_b1, z_b1,
                  is0, iu0, ix0, iah0, iaf0, is1, iu1, ix1, iah1, iaf1,
                  acc, s00, s01, s02, s03, s10, s11, s12, s13):
    cid = lax.axis_index("c")
    sid = lax.axis_index("s")

    bufs = ((h_b0, c_b0, z_b0, is0, iu0, ix0, iah0, iaf0,
             (s00, s01, s02, s03)),
            (h_b1, c_b1, z_b1, is1, iu1, ix1, iah1, iaf1,
             (s10, s11, s12, s13)))

    pltpu.sync_copy(tpk, tpk_v)
    pltpu.sync_copy(rel, rel_v)

    z16 = jnp.zeros((16,), jnp.float32)
    for r in range(ZR):
        for k in range(H // 16):
            zbuf[r, pl.ds(k * 16, 16)] = z16

    lane = lax.iota(jnp.int32, 16)

    def pre_blk(b, carry):
        base = sid * EPT + b * EB
        pltpu.sync_copy(esrc.at[pl.ds(base, EB)], cpk.at[pl.ds(0, EB)])
        pltpu.sync_copy(edst.at[pl.ds(base, EB)], cpk.at[pl.ds(EB, EB)])

        def pre_grp(g, cg):
            s16 = cpk[pl.ds(g * 16, 16)]
            d16 = cpk[pl.ds(EB + g * 16, 16)]
            tsw = plsc.load_gather(tpk_v, [lax.shift_right_logical(s16, 2)])
            ts = lax.shift_right_logical(tsw, (s16 & 3) * 8) & 0xFF
            tdw = plsc.load_gather(tpk_v, [lax.shift_right_logical(d16, 2)])
            td = lax.shift_right_logical(tdw, (d16 & 3) * 8) & 0xFF
            m16 = plsc.load_gather(rel_v, [ts * T + td])
            epk[pl.ds(b * EB + g * 16, 16)] = (
                s16 | lax.shift_left(d16, 14) | lax.shift_left(m16, 28))
            return cg

        lax.fori_loop(0, NG16, pre_grp, 0)
        return carry

    lax.fori_loop(0, EPT // EB, pre_blk, 0)

    sc_lo = cid * (CH_PER_SC * CHUNK)

    def hfil(g, cnt):
        v16 = epk[pl.ds(g * 16, 16)]
        d16 = lax.shift_right_logical(v16, 14) & 0x3FFF
        msk = (d16 >= sc_lo) & (d16 < sc_lo + CH_PER_SC * CHUNK)
        plsc.store_compressed(epk.at[pl.ds(cnt, 16)], v16, mask=msk)
        return cnt + jnp.sum(msk.astype(jnp.int32))

    ecnt = lax.fori_loop(0, EPT // 16, hfil, jnp.int32(0))
    eg16 = (ecnt + 15) // 16

    def pass_body(p, carry):
        chunk = cid * CH_PER_SC + p
        lo = chunk * CHUNK

        def zcp(z, cz):
            pltpu.sync_copy(zbuf, acc.at[pl.ds(sid * (ACC_ALLOC // 16) + z * ZR, ZR)])
            return cz
        lax.fori_loop(0, ACC_ALLOC // 16 // ZR, zcp, 0)
        plsc.subcore_barrier()

        def cgrp(g, cnt):
            v16 = epk[pl.ds(g * 16, 16)]
            d16 = lax.shift_right_logical(v16, 14) & 0x3FFF
            msk = (d16 >= lo) & (d16 < lo + CHUNK) & ((g * 16 + lane) < ecnt)
            plsc.store_compressed(cpk.at[pl.ds(cnt, 16)], v16, mask=msk)
            return cnt + jnp.sum(msk.astype(jnp.int32))

        cnt = lax.fori_loop(0, eg16, cgrp, jnp.int32(0))

        ngrp = (cnt + G - 1) // G
        trash = ACC_ROWS + sid

        def mkidx_and_fire(g, par):
            h_b, c_b, z_b, i_s, i_u, i_x, i_ah, i_af, sems = bufs[par]
            off = g * G

            def mkidx(q, cq):
                e0 = off + q * 16
                valid = (e0 + lane) < cnt
                v16 = cpk[pl.ds(e0, 16)]
                sv = v16 & 0x3FFF
                dv = lax.shift_right_logical(v16, 14) & 0x3FFF
                mv = lax.shift_right_logical(v16, 28)
                i_s[pl.ds(q * 16, 16)] = jnp.where(valid, sv, 0)
                i_u[pl.ds(q * 16, 16)] = jnp.where(valid, mv * N + sv, 0)
                i_x[pl.ds(q * 16, 16)] = jnp.where(valid, dv, 0)
                dl = dv - lo
                i_ah[pl.ds(q * 16, 16)] = jnp.where(valid, (mv + 1) * CHUNK + dl, trash)
                i_af[pl.ds(q * 16, 16)] = jnp.where(valid, dl, trash)
                return cq
            lax.fori_loop(0, G // 16, mkidx, 0)

            pltpu.async_copy(htab.at[i_s], h_b, sems[0])
            pltpu.async_copy(ctab.at[i_s], c_b, sems[1])
            pltpu.async_copy(xftab.at[i_x], z_b, sems[2])

        def fire_ufadd(par):
            h_b, c_b, z_b, i_s, i_u, i_x, i_ah, i_af, sems = bufs[par]
            pltpu.make_async_copy(xftab.at[i_x], z_b, sems[2]).wait()
            pltpu.async_copy(uftab.at[i_u], z_b, sems[3], add=True)

        def consume(par):
            h_b, c_b, z_b, i_s, i_u, i_x, i_ah, i_af, sems = bufs[par]
            pltpu.make_async_copy(htab.at[i_s], h_b, sems[0]).wait()
            pltpu.make_async_copy(ctab.at[i_s], c_b, sems[1]).wait()
            pltpu.make_async_copy(uftab.at[i_u], z_b, sems[3]).wait()

            @plsc.parallel_loop(0, G, unroll=2)
            def frow(r):
                for k in range(H // 16):
                    z = z_b[r, pl.ds(k * 16, 16)]
                    f = 1.0 / (1.0 + jnp.exp(-z))
                    z_b[r, pl.ds(k * 16, 16)] = c_b[r, pl.ds(k * 16, 16)] * f

            pltpu.sync_copy(h_b, acc.at[i_ah], add=True)
            pltpu.sync_copy(z_b, acc.at[i_af], add=True)

        @pl.when(ngrp > 0)
        def _prologue():
            mkidx_and_fire(0, 0)
            fire_ufadd(0)

        def pair(i, carry2):
            for par in (0, 1):
                g = 2 * i + par

                @pl.when(g + 1 < ngrp)
                def _fire_next():
                    mkidx_and_fire(g + 1, 1 - par)

                @pl.when(g < ngrp)
                def _consume():
                    consume(par)

                @pl.when(g + 1 < ngrp)
                def _fire_ufadd_next():
                    fire_ufadd(1 - par)
            return carry2

        lax.fori_loop(0, (ngrp + 1) // 2, pair, 0)
        plsc.subcore_barrier()

        rg = sid // 2
        kh = sid % 2
        row0 = rg * CHUNK + kh * (CHUNK // 2)
        dst0 = rg * NPAD + lo + kh * (CHUNK // 2)
        pltpu.sync_copy(acc.at[pl.ds(row0, CHUNK // 2)],
                        out.at[pl.ds(dst0, CHUNK // 2)])
        plsc.subcore_barrier()
        return carry

    lax.fori_loop(0, CH_PER_SC, pass_body, 0)


_sc_edge = functools.partial(
    pl.kernel,
    mesh=plsc.VectorSubcoreMesh(core_axis_name="c", subcore_axis_name="s"),
    out_type=jax.ShapeDtypeStruct(((M + 1) * NPAD, H), jnp.float32),
    compiler_params=pltpu.CompilerParams(needs_layout_passes=False),
    scratch_types=(
        [
            pltpu.VMEM((TPKN,), jnp.int32),
            pltpu.VMEM((T * T,), jnp.int32),
            pltpu.VMEM((EPT,), jnp.int32),
            pltpu.VMEM((EPT + 16,), jnp.int32),
            pltpu.VMEM((ZR, H), jnp.float32),
        ]
        + [pltpu.VMEM((G, H), jnp.float32)] * 6
        + [pltpu.VMEM((G,), jnp.int32)] * 10
        + [pltpu.VMEM_SHARED((ACC_ALLOC, H), jnp.float32)]
        + [pltpu.SemaphoreType.DMA] * 8
    ),
)(_sc_edge_body)


def kernel(x, edge_index, type_id, rel_table, W_iou, b_iou, W_f, b_f, U_iou, U_f):
    R = 400
    grid = (N // R,)

    xiou, xf, h1, c1, ufn = pl.pallas_call(
        _tc_pre_body,
        grid=grid,
        in_specs=[
            pl.BlockSpec((R, X), lambda i: (i, 0)),
            pl.BlockSpec((X, 3 * H), lambda i: (0, 0)),
            pl.BlockSpec((1, 3 * H), lambda i: (0, 0)),
            pl.BlockSpec((X, H), lambda i: (0, 0)),
            pl.BlockSpec((1, H), lambda i: (0, 0)),
            pl.BlockSpec((M, H, H), lambda i: (0, 0, 0)),
        ],
        out_specs=[
            pl.BlockSpec((R, 3 * H), lambda i: (i, 0)),
            pl.BlockSpec((R, H), lambda i: (i, 0)),
            pl.BlockSpec((R, H), lambda i: (i, 0)),
            pl.BlockSpec((R, H), lambda i: (i, 0)),
            pl.BlockSpec((M, R, H), lambda i: (0, i, 0)),
        ],
        out_shape=[
            jax.ShapeDtypeStruct((N, 3 * H), jnp.float32),
            jax.ShapeDtypeStruct((N, H), jnp.float32),
            jax.ShapeDtypeStruct((N, H), jnp.float32),
            jax.ShapeDtypeStruct((N, H), jnp.float32),
            jax.ShapeDtypeStruct((M, N, H), jnp.float32),
        ],
    )(x, W_iou.T, b_iou, W_f.T, b_f, U_f)

    tpk = jnp.pad(
        jax.lax.bitcast_convert_type(
            type_id.astype(jnp.uint8).reshape(N // 4, 4), jnp.int32),
        (0, TPKN - N // 4))
    out_all = _sc_edge(
        edge_index[0], edge_index[1], tpk, rel_table.reshape(T * T),
        h1, c1, ufn.reshape(M * N, H), xf)

    sums = out_all.reshape(M + 1, NPAD, H)

    h2, c2 = pl.pallas_call(
        _tc_post_body,
        grid=grid,
        in_specs=[
            pl.BlockSpec((M + 1, R, H), lambda i: (0, i, 0)),
            pl.BlockSpec((M, H, 3 * H), lambda i: (0, 0, 0)),
            pl.BlockSpec((R, 3 * H), lambda i: (i, 0)),
        ],
        out_specs=[
            pl.BlockSpec((R, H), lambda i: (i, 0)),
            pl.BlockSpec((R, H), lambda i: (i, 0)),
        ],
        out_shape=[
            jax.ShapeDtypeStruct((N, H), jnp.float32),
            jax.ShapeDtypeStruct((N, H), jnp.float32),
        ],
    )(sums, U_iou, xiou)
    return h2, c2


# speedup vs baseline: 1.2026x; 1.2026x over previous
import functools

import jax
import jax.numpy as jnp
from jax import lax
from jax.experimental import pallas as pl
from jax.experimental.pallas import tpu as pltpu
from jax.experimental.pallas import tpu_sc as plsc

N = 10000
E = 160000
X = 128
H = 128
M = 7
T = 16

NCHUNK = 20
NPAD = 10240
CHUNK = NPAD // NCHUNK
CH_PER_SC = NCHUNK // 2
ACC_ROWS = (M + 1) * CHUNK
ACC_ALLOC = ACC_ROWS + 512
EPT = E // 16
EB = 2000
NG16 = EB // 16
G = 64
ZR = 16
TPKN = N // 4 + 16


def _tc_pre_body(x_ref, wiou_ref, biou_ref, wf_ref, bf_ref, uf_ref,
                 xiou_ref, xf_ref, h_ref, c_ref, ufn_ref):
    xb = x_ref[...]
    xiou = jnp.dot(xb, wiou_ref[...], preferred_element_type=jnp.float32) + biou_ref[...]
    xf = jnp.dot(xb, wf_ref[...], preferred_element_type=jnp.float32) + bf_ref[...]
    i = jax.nn.sigmoid(xiou[:, :H])
    o = jax.nn.sigmoid(xiou[:, H:2 * H])
    u = jnp.tanh(xiou[:, 2 * H:])
    c1 = i * u
    h1 = o * jnp.tanh(c1)
    xiou_ref[...] = xiou
    xf_ref[...] = xf
    h_ref[...] = h1
    c_ref[...] = c1
    for m in range(M):
        ufn_ref[m] = jnp.dot(h1, uf_ref[m], preferred_element_type=jnp.float32)


def _tc_post_body(sums_ref, uiou_ref, xiou_ref, h_ref, c_ref):
    acc = xiou_ref[...]
    for m in range(M):
        acc = acc + jnp.dot(sums_ref[m + 1], uiou_ref[m], preferred_element_type=jnp.float32)
    i = jax.nn.sigmoid(acc[:, :H])
    o = jax.nn.sigmoid(acc[:, H:2 * H])
    u = jnp.tanh(acc[:, 2 * H:])
    c2 = i * u + sums_ref[0]
    h2 = o * jnp.tanh(c2)
    h_ref[...] = h2
    c_ref[...] = c2


def _sc_edge_body(esrc, edst, tpk, rel, htab, ctab, uftab, xftab,
                  out,
                  tpk_v, rel_v, epk, cpk, zbuf,
                  h_b0, c_b0, uf_b0, xf_b0, h_b1, c_b1, uf_b1, xf_b1,
                  is0, iu0, ix0, iah0, iaf0, is1, iu1, ix1, iah1, iaf1,
                  acc, s00, s01, s02, s03, t00, t01, s10, s11, s12, s13, t10, t11):
    cid = lax.axis_index("c")
    sid = lax.axis_index("s")

    bufs = ((h_b0, c_b0, uf_b0, xf_b0, is0, iu0, ix0, iah0, iaf0,
             (s00, s01, s02, s03, t00, t01)),
            (h_b1, c_b1, uf_b1, xf_b1, is1, iu1, ix1, iah1, iaf1,
             (s10, s11, s12, s13, t10, t11)))

    pltpu.sync_copy(tpk, tpk_v)
    pltpu.sync_copy(rel, rel_v)

    z16 = jnp.zeros((16,), jnp.float32)
    for r in range(ZR):
        for k in range(H // 16):
            zbuf[r, pl.ds(k * 16, 16)] = z16

    lane = lax.iota(jnp.int32, 16)

    def pre_blk(b, carry):
        base = sid * EPT + b * EB
        pltpu.sync_copy(esrc.at[pl.ds(base, EB)], cpk.at[pl.ds(0, EB)])
        pltpu.sync_copy(edst.at[pl.ds(base, EB)], cpk.at[pl.ds(EB, EB)])

        def pre_grp(g, cg):
            s16 = cpk[pl.ds(g * 16, 16)]
            d16 = cpk[pl.ds(EB + g * 16, 16)]
            tsw = plsc.load_gather(tpk_v, [lax.shift_right_logical(s16, 2)])
            ts = lax.shift_right_logical(tsw, (s16 & 3) * 8) & 0xFF
            tdw = plsc.load_gather(tpk_v, [lax.shift_right_logical(d16, 2)])
            td = lax.shift_right_logical(tdw, (d16 & 3) * 8) & 0xFF
            m16 = plsc.load_gather(rel_v, [ts * T + td])
            epk[pl.ds(b * EB + g * 16, 16)] = (
                s16 | lax.shift_left(d16, 14) | lax.shift_left(m16, 28))
            return cg

        lax.fori_loop(0, NG16, pre_grp, 0)
        return carry

    lax.fori_loop(0, EPT // EB, pre_blk, 0)

    sc_lo = cid * (CH_PER_SC * CHUNK)

    def hfil(g, cnt):
        v16 = epk[pl.ds(g * 16, 16)]
        d16 = lax.shift_right_logical(v16, 14) & 0x3FFF
        msk = (d16 >= sc_lo) & (d16 < sc_lo + CH_PER_SC * CHUNK)
        plsc.store_compressed(epk.at[pl.ds(cnt, 16)], v16, mask=msk)
        return cnt + jnp.sum(msk.astype(jnp.int32))

    ecnt = lax.fori_loop(0, EPT // 16, hfil, jnp.int32(0))
    eg16 = (ecnt + 15) // 16

    def pass_body(p, carry):
        chunk = cid * CH_PER_SC + p
        lo = chunk * CHUNK

        def zcp(z, cz):
            pltpu.sync_copy(zbuf, acc.at[pl.ds(sid * (ACC_ALLOC // 16) + z * ZR, ZR)])
            return cz
        lax.fori_loop(0, ACC_ALLOC // 16 // ZR, zcp, 0)
        plsc.subcore_barrier()

        def cgrp(g, cnt):
            v16 = epk[pl.ds(g * 16, 16)]
            d16 = lax.shift_right_logical(v16, 14) & 0x3FFF
            msk = (d16 >= lo) & (d16 < lo + CHUNK) & ((g * 16 + lane) < ecnt)
            plsc.store_compressed(cpk.at[pl.ds(cnt, 16)], v16, mask=msk)
            return cnt + jnp.sum(msk.astype(jnp.int32))

        cnt = lax.fori_loop(0, eg16, cgrp, jnp.int32(0))

        ngrp = (cnt + G - 1) // G
        trash = ACC_ROWS + sid

        def mkidx_and_fire(g, par):
            h_b, c_b, uf_b, xf_b, i_s, i_u, i_x, i_ah, i_af, sems = bufs[par]
            off = g * G

            @pl.when(g >= 2)
            def _drain():
                pltpu.make_async_copy(h_b, acc.at[i_ah], sems[4]).wait()
                pltpu.make_async_copy(xf_b, acc.at[i_af], sems[5]).wait()

            def mkidx(q, cq):
                e0 = off + q * 16
                valid = (e0 + lane) < cnt
                v16 = cpk[pl.ds(e0, 16)]
                sv = v16 & 0x3FFF
                dv = lax.shift_right_logical(v16, 14) & 0x3FFF
                mv = lax.shift_right_logical(v16, 28)
                i_s[pl.ds(q * 16, 16)] = jnp.where(valid, sv, 0)
                i_u[pl.ds(q * 16, 16)] = jnp.where(valid, mv * N + sv, 0)
                i_x[pl.ds(q * 16, 16)] = jnp.where(valid, dv, 0)
                dl = dv - lo
                i_ah[pl.ds(q * 16, 16)] = jnp.where(valid, (mv + 1) * CHUNK + dl, trash)
                i_af[pl.ds(q * 16, 16)] = jnp.where(valid, dl, trash)
                return cq
            lax.fori_loop(0, G // 16, mkidx, 0)

            pltpu.async_copy(htab.at[i_s], h_b, sems[0])
            pltpu.async_copy(ctab.at[i_s], c_b, sems[1])
            pltpu.async_copy(uftab.at[i_u], uf_b, sems[2])
            pltpu.async_copy(xftab.at[i_x], xf_b, sems[3])

        def consume(par):
            h_b, c_b, uf_b, xf_b, i_s, i_u, i_x, i_ah, i_af, sems = bufs[par]
            pltpu.make_async_copy(htab.at[i_s], h_b, sems[0]).wait()
            pltpu.make_async_copy(ctab.at[i_s], c_b, sems[1]).wait()
            pltpu.make_async_copy(uftab.at[i_u], uf_b, sems[2]).wait()
            pltpu.make_async_copy(xftab.at[i_x], xf_b, sems[3]).wait()

            pltpu.async_copy(h_b, acc.at[i_ah], sems[4], add=True)

            @plsc.parallel_loop(0, G, unroll=2)
            def frow(r):
                for k in range(H // 16):
                    z = xf_b[r, pl.ds(k * 16, 16)] + uf_b[r, pl.ds(k * 16, 16)]
                    f = 1.0 / (1.0 + jnp.exp(-z))
                    xf_b[r, pl.ds(k * 16, 16)] = c_b[r, pl.ds(k * 16, 16)] * f

            pltpu.async_copy(xf_b, acc.at[i_af], sems[5], add=True)

        @pl.when(ngrp > 0)
        def _prologue():
            mkidx_and_fire(0, 0)

        def pair(i, carry2):
            for par in (0, 1):
                g = 2 * i + par

                @pl.when(g + 1 < ngrp)
                def _fire_next():
                    mkidx_and_fire(g + 1, 1 - par)

                @pl.when(g < ngrp)
                def _consume():
                    consume(par)
            return carry2

        lax.fori_loop(0, (ngrp + 1) // 2, pair, 0)

        for par in (0, 1):
            h_b, c_b, uf_b, xf_b, i_s, i_u, i_x, i_ah, i_af, sems = bufs[par]

            @pl.when(ngrp > par)
            def _final_drain():
                pltpu.make_async_copy(h_b, acc.at[i_ah], sems[4]).wait()
                pltpu.make_async_copy(xf_b, acc.at[i_af], sems[5]).wait()

        plsc.subcore_barrier()

        rg = sid // 2
        kh = sid % 2
        row0 = rg * CHUNK + kh * (CHUNK // 2)
        dst0 = rg * NPAD + lo + kh * (CHUNK // 2)
        pltpu.sync_copy(acc.at[pl.ds(row0, CHUNK // 2)],
                        out.at[pl.ds(dst0, CHUNK // 2)])
        plsc.subcore_barrier()
        return carry

    lax.fori_loop(0, CH_PER_SC, pass_body, 0)


_sc_edge = functools.partial(
    pl.kernel,
    mesh=plsc.VectorSubcoreMesh(core_axis_name="c", subcore_axis_name="s"),
    out_type=jax.ShapeDtypeStruct(((M + 1) * NPAD, H), jnp.float32),
    compiler_params=pltpu.CompilerParams(needs_layout_passes=False),
    scratch_types=(
        [
            pltpu.VMEM((TPKN,), jnp.int32),
            pltpu.VMEM((T * T,), jnp.int32),
            pltpu.VMEM((EPT,), jnp.int32),
            pltpu.VMEM((EPT + 16,), jnp.int32),
            pltpu.VMEM((ZR, H), jnp.float32),
        ]
        + [pltpu.VMEM((G, H), jnp.float32)] * 8
        + [pltpu.VMEM((G,), jnp.int32)] * 10
        + [pltpu.VMEM_SHARED((ACC_ALLOC, H), jnp.float32)]
        + [pltpu.SemaphoreType.DMA] * 12
    ),
)(_sc_edge_body)


def kernel(x, edge_index, type_id, rel_table, W_iou, b_iou, W_f, b_f, U_iou, U_f):
    R = 400
    grid = (N // R,)

    xiou, xf, h1, c1, ufn = pl.pallas_call(
        _tc_pre_body,
        grid=grid,
        in_specs=[
            pl.BlockSpec((R, X), lambda i: (i, 0)),
            pl.BlockSpec((X, 3 * H), lambda i: (0, 0)),
            pl.BlockSpec((1, 3 * H), lambda i: (0, 0)),
            pl.BlockSpec((X, H), lambda i: (0, 0)),
            pl.BlockSpec((1, H), lambda i: (0, 0)),
            pl.BlockSpec((M, H, H), lambda i: (0, 0, 0)),
        ],
        out_specs=[
            pl.BlockSpec((R, 3 * H), lambda i: (i, 0)),
            pl.BlockSpec((R, H), lambda i: (i, 0)),
            pl.BlockSpec((R, H), lambda i: (i, 0)),
            pl.BlockSpec((R, H), lambda i: (i, 0)),
            pl.BlockSpec((M, R, H), lambda i: (0, i, 0)),
        ],
        out_shape=[
            jax.ShapeDtypeStruct((N, 3 * H), jnp.float32),
            jax.ShapeDtypeStruct((N, H), jnp.float32),
            jax.ShapeDtypeStruct((N, H), jnp.float32),
            jax.ShapeDtypeStruct((N, H), jnp.float32),
            jax.ShapeDtypeStruct((M, N, H), jnp.float32),
        ],
    )(x, W_iou.T, b_iou, W_f.T, b_f, U_f)

    tpk = jnp.pad(
        jax.lax.bitcast_convert_type(
            type_id.astype(jnp.uint8).reshape(N // 4, 4), jnp.int32),
        (0, TPKN - N // 4))
    out_all = _sc_edge(
        edge_index[0], edge_index[1], tpk, rel_table.reshape(T * T),
        h1, c1, ufn.reshape(M * N, H), xf)

    sums = out_all.reshape(M + 1, NPAD, H)

    h2, c2 = pl.pallas_call(
        _tc_post_body,
        grid=grid,
        in_specs=[
            pl.BlockSpec((M + 1, R, H), lambda i: (0, i, 0)),
            pl.BlockSpec((M, H, 3 * H), lambda i: (0, 0, 0)),
            pl.BlockSpec((R, 3 * H), lambda i: (i, 0)),
        ],
        out_specs=[
            pl.BlockSpec((R, H), lambda i: (i, 0)),
            pl.BlockSpec((R, H), lambda i: (i, 0)),
        ],
        out_shape=[
            jax.ShapeDtypeStruct((N, H), jnp.float32),
            jax.ShapeDtypeStruct((N, H), jnp.float32),
        ],
    )(sums, U_iou, xiou)
    return h2, c2
